# baseline (speedup 1.0000x reference)
import functools

import jax
import jax.numpy as jnp
from jax import lax
from jax.experimental import pallas as pl
from jax.experimental.pallas import tpu as pltpu
from jax.experimental.pallas import tpu_sc as plsc

N = 10000
E = 320000
D = 128
NW = 32
NB = 320
NP = NW * NB
CHUNK = 1280
NCHUNK = E // CHUNK
NVEC = CHUNK // 16
SB = 128
NBLK = NP // SB
CAPB = 16384

_mesh = plsc.VectorSubcoreMesh(
    core_axis_name="c", subcore_axis_name="s", num_cores=2, num_subcores=16
)


@functools.partial(
    pl.kernel,
    out_type=(
        jax.ShapeDtypeStruct((NP, D), jnp.float32),
        jax.ShapeDtypeStruct((NP * 16,), jnp.float32),
    ),
    mesh=_mesh,
    compiler_params=pltpu.CompilerParams(needs_layout_passes=False),
    scratch_types=[
        pltpu.VMEM((NB + 8, D), jnp.float32),
        pltpu.VMEM(((NB + 1) * 16,), jnp.float32),
        pltpu.VMEM((NB * 16,), jnp.float32),
        pltpu.VMEM((CHUNK,), jnp.int32),
        pltpu.VMEM((CHUNK,), jnp.int32),
        pltpu.VMEM((CHUNK,), jnp.int32),
        pltpu.VMEM((CHUNK,), jnp.int32),
        pltpu.VMEM((CAPB + 16,), jnp.int32),
        pltpu.VMEM((CAPB + 16,), jnp.int32),
        pltpu.VMEM((SB, 2 * D), jnp.float32),
        pltpu.SemaphoreType.DMA,
        pltpu.SemaphoreType.DMA,
        pltpu.SemaphoreType.DMA,
    ],
)
def _pointconv_sc(xe_hbm, x_hbm, posp_hbm, src_hbm, dst_hbm,
                  outx_hbm, outp_hbm,
                  accx, accp, posblk, sbuf0, sbuf1, dbuf0, dbuf1,
                  store, stage, xblk, sem, esem0, esem1):
    wid = lax.axis_index("s") * 2 + lax.axis_index("c")
    base = wid * NB

    pltpu.sync_copy(x_hbm.at[pl.ds(base, NB)], accx.at[pl.ds(0, NB)])
    pltpu.sync_copy(posp_hbm.at[pl.ds(base * 16, NB * 16)], posblk)

    def seedp(j, carry):
        o = pl.multiple_of(j * 16, 16)
        accp[pl.ds(o, 16)] = posblk[pl.ds(o, 16)]
        return carry

    lax.fori_loop(0, NB, seedp, 0)

    iota16 = lax.iota(jnp.int32, 16)

    def round_body(carry):
        r, _ = carry
        lo_w = r * CAPB

        sb = (sbuf0, sbuf1)
        db = (dbuf0, dbuf1)
        esems = (esem0, esem1)

        def fire(c, q):
            e0 = pl.multiple_of(c * CHUNK, CHUNK)
            pltpu.async_copy(src_hbm.at[pl.ds(e0, CHUNK)], sb[q], esems[q])
            pltpu.async_copy(dst_hbm.at[pl.ds(e0, CHUNK)], db[q], esems[q])

        def wait_q(q):
            pltpu.make_async_copy(src_hbm.at[pl.ds(0, CHUNK)], sb[q],
                                  esems[q]).wait()
            pltpu.make_async_copy(dst_hbm.at[pl.ds(0, CHUNK)], db[q],
                                  esems[q]).wait()

        def scan_chunk(q, cnt_v):
            def scan_body(i, cnt_v2):
                o = pl.multiple_of(i * 16, 16)
                dvec = db[q][pl.ds(o, 16)]
                m = (dvec >= base) & (dvec < base + NB)
                svec = sb[q][pl.ds(o, 16)]
                pref = plsc.cumsum(m.astype(jnp.int32))
                posv = cnt_v2 + pref - 1
                mst = m & (posv >= lo_w) & (posv < lo_w + CAPB)
                packed = jnp.left_shift(svec, 9) | (dvec - base)
                plsc.store_scatter(store, [posv - lo_w], packed, mask=mst)
                return cnt_v2 + plsc.all_reduce_population_count(m)

            return lax.fori_loop(0, NVEC, scan_body, cnt_v)

        fire(0, 0)

        def pair_body(c2, cnt_v):
            c = c2 * 2
            fire(c + 1, 1)
            wait_q(0)
            cnt_v = scan_chunk(0, cnt_v)
            fire(jnp.minimum(c + 2, NCHUNK - 1), 0)
            wait_q(1)
            return scan_chunk(1, cnt_v)

        cnt_v = lax.fori_loop(0, NCHUNK // 2, pair_body,
                              jnp.zeros((16,), jnp.int32))
        wait_q(0)
        total = jnp.max(cnt_v)
        mm = jnp.minimum(total - lo_w, CAPB)

        fl = (mm // 16) * 16
        idxv = fl + iota16
        plsc.store_scatter(store, [idxv], jnp.full((16,), NB, jnp.int32),
                           mask=idxv >= mm)
        mmp = ((mm + 15) // 16) * 16

        def blk_body(b, carry2):
            sb0 = pl.multiple_of(b * SB, SB)
            cp = pltpu.async_copy(xe_hbm.at[pl.ds(sb0, SB)], xblk, sem)

            plo = jnp.left_shift(sb0, 9)
            phi = jnp.left_shift(sb0 + SB, 9)

            def rs_body(i, bcnt_v):
                o = pl.multiple_of(i * 16, 16)
                pk = store[pl.ds(o, 16)]
                mb = (pk >= plo) & (pk < phi)
                popc = plsc.all_reduce_population_count(mb)

                @pl.when(popc[0] > 0)
                def _():
                    pref = plsc.cumsum(mb.astype(jnp.int32))
                    posv = bcnt_v + pref - 1
                    plsc.store_scatter(stage, [posv], pk, mask=mb)

                return bcnt_v + popc

            bcnt_v = lax.fori_loop(0, mmp // 16, rs_body,
                                   jnp.zeros((16,), jnp.int32))
            bcnt = jnp.max(bcnt_v)
            bfl = (bcnt // 16) * 16
            bidx = bfl + iota16
            dummy = jnp.full((16,), (sb0 << 9) | NB, jnp.int32)
            plsc.store_scatter(stage, [bidx], dummy, mask=bidx >= bcnt)
            bp = ((bcnt + 15) // 16) * 16

            cp.wait()

            def grp_body(t, carry3):
                o = pl.multiple_of(t * 16, 16)
                pkv = stage[pl.ds(o, 16)]
                dv = pkv & 511
                sv = jnp.right_shift(pkv, 9) - sb0
                for l in range(16):
                    d = dv[l]
                    s = sv[l]
                    dp = d * 16
                    mx = [jnp.maximum(accx[d, pl.ds(k * 16, 16)],
                                      xblk[s, pl.ds(k * 16, 16)])
                          for k in range(D // 16)]
                    mp = jnp.maximum(accp[pl.ds(dp, 16)],
                                     xblk[s, pl.ds(D, 16)])
                    for k in range(D // 16):
                        accx[d, pl.ds(k * 16, 16)] = mx[k]
                    accp[pl.ds(dp, 16)] = mp
                return carry3

            lax.fori_loop(0, bp // 16, grp_body, 0)
            return carry2

        lax.fori_loop(0, NBLK, blk_body, 0)
        return r + 1, total

    def round_cond(carry):
        r, total = carry
        return (r == 0) | (r * CAPB < total)

    lax.while_loop(round_cond, round_body, (jnp.int32(0), jnp.int32(0)))

    def fin(j, carry):
        o = pl.multiple_of(j * 16, 16)
        accp[pl.ds(o, 16)] = accp[pl.ds(o, 16)] - posblk[pl.ds(o, 16)]
        return carry

    lax.fori_loop(0, NB, fin, 0)

    pltpu.sync_copy(accx.at[pl.ds(0, NB)], outx_hbm.at[pl.ds(base, NB)])
    pltpu.sync_copy(accp.at[pl.ds(0, NB * 16)],
                    outp_hbm.at[pl.ds(base * 16, NB * 16)])


def kernel(x, pos, edge_index):
    xf = x.astype(jnp.float32)
    posf = pos.astype(jnp.float32)
    xpad = jnp.zeros((NP, D), jnp.float32).at[:N].set(xf)
    xe = jnp.zeros((NP, 2 * D), jnp.float32)
    xe = xe.at[:N, :D].set(xf).at[:N, D:D + 3].set(posf)
    posp = jnp.zeros((NP, 16), jnp.float32).at[:N, :3].set(posf).reshape(-1)
    ei = edge_index.astype(jnp.int32)
    outx, outp = _pointconv_sc(xe, xpad, posp, ei[0], ei[1])
    return jnp.concatenate([outx[:N], outp.reshape(NP, 16)[:N, :3]], axis=1)


# speedup vs baseline: 1.4928x; 1.4928x over previous
import functools

import jax
import jax.numpy as jnp
from jax import lax
from jax.experimental import pallas as pl
from jax.experimental.pallas import tpu as pltpu
from jax.experimental.pallas import tpu_sc as plsc

N = 10000
E = 320000
D = 128
NW = 32
NB = 320
NP = NW * NB
CHUNK = 1280
NCHUNK = E // CHUNK
NVEC = CHUNK // 16
SB = 128
NBLK = NP // SB
CAPB = 16384

_mesh = plsc.VectorSubcoreMesh(
    core_axis_name="c", subcore_axis_name="s", num_cores=2, num_subcores=16
)


@functools.partial(
    pl.kernel,
    out_type=(
        jax.ShapeDtypeStruct((NP, D), jnp.float32),
        jax.ShapeDtypeStruct((NP * 16,), jnp.float32),
    ),
    mesh=_mesh,
    compiler_params=pltpu.CompilerParams(needs_layout_passes=False),
    scratch_types=[
        pltpu.VMEM((NB + 8, D), jnp.float32),
        pltpu.VMEM(((NB + 1) * 16,), jnp.float32),
        pltpu.VMEM((NB * 16,), jnp.float32),
        pltpu.VMEM((CHUNK,), jnp.int32),
        pltpu.VMEM((CHUNK,), jnp.int32),
        pltpu.VMEM((CHUNK,), jnp.int32),
        pltpu.VMEM((CHUNK,), jnp.int32),
        pltpu.VMEM((CAPB + 16,), jnp.int32),
        pltpu.VMEM((CAPB + 16,), jnp.int32),
        pltpu.VMEM((SB, 2 * D), jnp.float32),
        pltpu.SemaphoreType.DMA,
        pltpu.SemaphoreType.DMA,
        pltpu.SemaphoreType.DMA,
    ],
)
def _pointconv_sc(xe_hbm, x_hbm, posp_hbm, src_hbm, dst_hbm,
                  outx_hbm, outp_hbm,
                  accx, accp, posblk, sbuf0, sbuf1, dbuf0, dbuf1,
                  store, stage, xblk, sem, esem0, esem1):
    wid = lax.axis_index("s") * 2 + lax.axis_index("c")
    base = wid * NB

    pltpu.sync_copy(x_hbm.at[pl.ds(base, NB)], accx.at[pl.ds(0, NB)])
    pltpu.sync_copy(posp_hbm.at[pl.ds(base * 16, NB * 16)], posblk)

    def seedp(j, carry):
        o = pl.multiple_of(j * 16, 16)
        accp[pl.ds(o, 16)] = posblk[pl.ds(o, 16)]
        return carry

    lax.fori_loop(0, NB, seedp, 0)

    iota16 = lax.iota(jnp.int32, 16)

    def round_body(carry):
        r, _ = carry
        lo_w = r * CAPB

        sb = (sbuf0, sbuf1)
        db = (dbuf0, dbuf1)
        esems = (esem0, esem1)

        def fire(c, q):
            e0 = pl.multiple_of(c * CHUNK, CHUNK)
            pltpu.async_copy(src_hbm.at[pl.ds(e0, CHUNK)], sb[q], esems[q])
            pltpu.async_copy(dst_hbm.at[pl.ds(e0, CHUNK)], db[q], esems[q])

        def wait_q(q):
            pltpu.make_async_copy(src_hbm.at[pl.ds(0, CHUNK)], sb[q],
                                  esems[q]).wait()
            pltpu.make_async_copy(dst_hbm.at[pl.ds(0, CHUNK)], db[q],
                                  esems[q]).wait()

        def scan_chunk(q, cnt_v):
            def scan_body(i, cnt_v2):
                o = pl.multiple_of(i * 16, 16)
                dvec = db[q][pl.ds(o, 16)]
                m = (dvec >= base) & (dvec < base + NB)
                svec = sb[q][pl.ds(o, 16)]
                pref = plsc.cumsum(m.astype(jnp.int32))
                posv = cnt_v2 + pref - 1
                mst = m & (posv >= lo_w) & (posv < lo_w + CAPB)
                packed = jnp.left_shift(svec, 9) | (dvec - base)
                plsc.store_scatter(store, [posv - lo_w], packed, mask=mst)
                return cnt_v2 + plsc.all_reduce_population_count(m)

            return lax.fori_loop(0, NVEC, scan_body, cnt_v)

        fire(0, 0)

        def pair_body(c2, cnt_v):
            c = c2 * 2
            fire(c + 1, 1)
            wait_q(0)
            cnt_v = scan_chunk(0, cnt_v)
            fire(jnp.minimum(c + 2, NCHUNK - 1), 0)
            wait_q(1)
            return scan_chunk(1, cnt_v)

        cnt_v = lax.fori_loop(0, NCHUNK // 2, pair_body,
                              jnp.zeros((16,), jnp.int32))
        wait_q(0)
        total = jnp.max(cnt_v)
        mm = jnp.minimum(total - lo_w, CAPB)

        fl = (mm // 16) * 16
        idxv = fl + iota16
        plsc.store_scatter(store, [idxv], jnp.full((16,), NB, jnp.int32),
                           mask=idxv >= mm)
        mmp = ((mm + 15) // 16) * 16

        def blk_body(b, carry2):
            sb0 = pl.multiple_of(b * SB, SB)
            cp = pltpu.async_copy(xe_hbm.at[pl.ds(sb0, SB)], xblk, sem)

            plo = jnp.left_shift(sb0, 9)
            phi = jnp.left_shift(sb0 + SB, 9)

            def rs_body(i, bcnt_v):
                o = pl.multiple_of(i * 16, 16)
                pk = store[pl.ds(o, 16)]
                mb = (pk >= plo) & (pk < phi)
                pref = plsc.cumsum(mb.astype(jnp.int32))
                posv = bcnt_v + pref - 1
                plsc.store_scatter(stage, [posv], pk, mask=mb)
                return bcnt_v + plsc.all_reduce_population_count(mb)

            bcnt_v = lax.fori_loop(0, mmp // 16, rs_body,
                                   jnp.zeros((16,), jnp.int32))
            bcnt = jnp.max(bcnt_v)
            bfl = (bcnt // 16) * 16
            bidx = bfl + iota16
            dummy = jnp.full((16,), (sb0 << 9) | NB, jnp.int32)
            plsc.store_scatter(stage, [bidx], dummy, mask=bidx >= bcnt)
            bp = ((bcnt + 15) // 16) * 16

            cp.wait()

            def grp_body(t, carry3):
                o = pl.multiple_of(t * 16, 16)
                pkv = stage[pl.ds(o, 16)]
                dv = pkv & 511
                sv = jnp.right_shift(pkv, 9) - sb0
                for l in range(16):
                    d = dv[l]
                    s = sv[l]
                    dp = d * 16
                    mx = [jnp.maximum(accx[d, pl.ds(k * 16, 16)],
                                      xblk[s, pl.ds(k * 16, 16)])
                          for k in range(D // 16)]
                    mp = jnp.maximum(accp[pl.ds(dp, 16)],
                                     xblk[s, pl.ds(D, 16)])
                    for k in range(D // 16):
                        accx[d, pl.ds(k * 16, 16)] = mx[k]
                    accp[pl.ds(dp, 16)] = mp
                return carry3

            lax.fori_loop(0, bp // 16, grp_body, 0)
            return carry2

        lax.fori_loop(0, NBLK, blk_body, 0)
        return r + 1, total

    def round_cond(carry):
        r, total = carry
        return (r == 0) | (r * CAPB < total)

    lax.while_loop(round_cond, round_body, (jnp.int32(0), jnp.int32(0)))

    def fin(j, carry):
        o = pl.multiple_of(j * 16, 16)
        accp[pl.ds(o, 16)] = accp[pl.ds(o, 16)] - posblk[pl.ds(o, 16)]
        return carry

    lax.fori_loop(0, NB, fin, 0)

    pltpu.sync_copy(accx.at[pl.ds(0, NB)], outx_hbm.at[pl.ds(base, NB)])
    pltpu.sync_copy(accp.at[pl.ds(0, NB * 16)],
                    outp_hbm.at[pl.ds(base * 16, NB * 16)])


def kernel(x, pos, edge_index):
    xf = x.astype(jnp.float32)
    posf = pos.astype(jnp.float32)
    xpad = jnp.zeros((NP, D), jnp.float32).at[:N].set(xf)
    xe = jnp.zeros((NP, 2 * D), jnp.float32)
    xe = xe.at[:N, :D].set(xf).at[:N, D:D + 3].set(posf)
    posp = jnp.zeros((NP, 16), jnp.float32).at[:N, :3].set(posf).reshape(-1)
    ei = edge_index.astype(jnp.int32)
    outx, outp = _pointconv_sc(xe, xpad, posp, ei[0], ei[1])
    return jnp.concatenate([outx[:N], outp.reshape(NP, 16)[:N, :3]], axis=1)
